# initial kernel scaffold (unmeasured)
import jax
import jax.numpy as jnp
from jax import lax
from jax.experimental import pallas as pl
from jax.experimental.pallas import tpu as pltpu

N_DEV = 4
SQ = 1024
SKV = 1024
D_MODEL = 1024
HQ_PER = 8
DH = 128
CHUNK = HQ_PER * DH
SCALE = 0.08838834764831843
BLK = 64


def kernel(x, Wq, K_ext, V_ext, Wo):
    x16 = x.astype(jnp.bfloat16)
    wq16 = Wq.astype(jnp.bfloat16)
    wo16 = Wo.astype(jnp.bfloat16)

    def body(x_ref, wq_ref, k_hbm, v_hbm, wo_ref, out_ref,
             comm, k_buf, v_buf, ctx_buf, mask_buf,
             send_sems, recv_sems, kv_sems):
        my = lax.axis_index("i")
        right = lax.rem(my + 1, N_DEV)
        left = lax.rem(my + 3, N_DEV)

        barrier = pltpu.get_barrier_semaphore()
        for nbr in (left, right):
            pl.semaphore_signal(
                barrier, inc=1,
                device_id=(nbr,), device_id_type=pl.DeviceIdType.MESH,
            )
        pl.semaphore_wait(barrier, 2)

        comm[0, 0] = wq_ref[...]
        comm[0, 1] = wo_ref[...]

        rb = lax.broadcasted_iota(jnp.int32, (SQ, SKV), 0) // BLK
        cb = lax.broadcasted_iota(jnp.int32, (SQ, SKV), 1) // BLK
        mask_buf[...] = jnp.where(cb <= rb, 0.0, -1e9)

        xb = x_ref[0]

        for h in range(N_DEV):
            c = lax.rem(my - h + N_DEV, N_DEV)

            kcp = pltpu.make_async_copy(
                k_hbm.at[my, :, pl.ds(c * HQ_PER, HQ_PER), :],
                k_buf, kv_sems.at[0])
            vcp = pltpu.make_async_copy(
                v_hbm.at[my, :, pl.ds(c * HQ_PER, HQ_PER), :],
                v_buf, kv_sems.at[1])
            kcp.start()
            vcp.start()

            if h < N_DEV - 1:
                rdma = pltpu.make_async_remote_copy(
                    src_ref=comm.at[h],
                    dst_ref=comm.at[h + 1],
                    send_sem=send_sems.at[h],
                    recv_sem=recv_sems.at[h],
                    device_id=(right,),
                    device_id_type=pl.DeviceIdType.MESH,
                )
                rdma.start()

            kcp.wait()
            vcp.wait()

            wq_c = comm[h, 0]
            q = lax.dot_general(
                xb, wq_c, (((1,), (0,)), ((), ())),
                preferred_element_type=jnp.float32)
            qb = q.astype(jnp.bfloat16)
            kb = k_buf[...].astype(jnp.bfloat16)
            vb = v_buf[...].astype(jnp.bfloat16)

            for hh in range(HQ_PER):
                qh = qb[:, hh * DH:(hh + 1) * DH]
                kh = kb[:, hh, :]
                s = lax.dot_general(
                    qh, kh, (((1,), (1,)), ((), ())),
                    preferred_element_type=jnp.float32)
                s = s * SCALE + mask_buf[...]
                m = jnp.max(s, axis=1, keepdims=True)
                w = jnp.exp(s - m)
                den = jnp.sum(w, axis=1, keepdims=True)
                wb = (w / den).astype(jnp.bfloat16)
                ctx = lax.dot_general(
                    wb, vb[:, hh, :], (((1,), (0,)), ((), ())),
                    preferred_element_type=jnp.float32)
                ctx_buf[:, hh * DH:(hh + 1) * DH] = ctx.astype(jnp.bfloat16)

            contrib = lax.dot_general(
                ctx_buf[...], comm[h, 1], (((1,), (0,)), ((), ())),
                preferred_element_type=jnp.float32)
            if h == 0:
                out_ref[0] = contrib
            else:
                out_ref[0] = out_ref[0] + contrib

            if h < N_DEV - 1:
                rdma.wait()

    return pl.pallas_call(
        body,
        out_shape=jax.ShapeDtypeStruct((1, SQ, D_MODEL), jnp.float32),
        in_specs=[
            pl.BlockSpec(memory_space=pltpu.VMEM),
            pl.BlockSpec(memory_space=pltpu.VMEM),
            pl.BlockSpec(memory_space=pltpu.ANY),
            pl.BlockSpec(memory_space=pltpu.ANY),
            pl.BlockSpec(memory_space=pltpu.VMEM),
        ],
        out_specs=pl.BlockSpec(memory_space=pltpu.VMEM),
        scratch_shapes=[
            pltpu.VMEM((N_DEV, 2, CHUNK, D_MODEL), jnp.bfloat16),
            pltpu.VMEM((SKV, HQ_PER, DH), jnp.float32),
            pltpu.VMEM((SKV, HQ_PER, DH), jnp.float32),
            pltpu.VMEM((SQ, CHUNK), jnp.bfloat16),
            pltpu.VMEM((SQ, SKV), jnp.float32),
            pltpu.SemaphoreType.DMA((N_DEV - 1,)),
            pltpu.SemaphoreType.DMA((N_DEV - 1,)),
            pltpu.SemaphoreType.DMA((2,)),
        ],
        compiler_params=pltpu.CompilerParams(collective_id=0),
    )(x16, wq16, K_ext, V_ext, wo16)


# baseline (device time: 187464 ns/iter reference)
import jax

jax.config.update("jax_compilation_cache_dir", "/tmp/scband_jax_cache")
jax.config.update("jax_persistent_cache_min_compile_time_secs", 0.0)
jax.config.update("jax_persistent_cache_min_entry_size_bytes", 0)

import jax.numpy as jnp
from jax import lax
from jax.experimental import pallas as pl
from jax.experimental.pallas import tpu as pltpu

N_DEV = 4
SQ = 1024
SKV = 1024
D_MODEL = 1024
HQ_PER = 8
DH = 128
CHUNK = HQ_PER * DH
SCALE = 0.08838834764831843
BLK = 64


def kernel(x, Wq, K_ext, V_ext, Wo):
    x16 = x.astype(jnp.bfloat16)
    wq16 = Wq.astype(jnp.bfloat16)
    wo16 = Wo.astype(jnp.bfloat16)

    def body(x_ref, wq_ref, k_hbm, v_hbm, wo_ref, out_ref,
             comm, k_buf, v_buf, ctx_buf, mask_buf,
             send_sems, recv_sems, kv_sems):
        my = lax.axis_index("i")
        right = lax.rem(my + 1, N_DEV)
        left = lax.rem(my + 3, N_DEV)

        barrier = pltpu.get_barrier_semaphore()
        for nbr in (left, right):
            pl.semaphore_signal(
                barrier, inc=1,
                device_id=(nbr,), device_id_type=pl.DeviceIdType.MESH,
            )
        pl.semaphore_wait(barrier, 2)

        comm[0, 0] = wq_ref[...]
        comm[0, 1] = wo_ref[...]

        rb = lax.broadcasted_iota(jnp.int32, (SQ, SKV), 0) // BLK
        cb = lax.broadcasted_iota(jnp.int32, (SQ, SKV), 1) // BLK
        mask_buf[...] = jnp.where(cb <= rb, 0.0, -1e9)

        xb = x_ref[0]

        for h in range(N_DEV):
            c = lax.rem(my - h + N_DEV, N_DEV)

            kcp = pltpu.make_async_copy(
                k_hbm.at[my, :, pl.ds(c * HQ_PER, HQ_PER), :],
                k_buf, kv_sems.at[0])
            vcp = pltpu.make_async_copy(
                v_hbm.at[my, :, pl.ds(c * HQ_PER, HQ_PER), :],
                v_buf, kv_sems.at[1])
            kcp.start()
            vcp.start()

            if h < N_DEV - 1:
                rdma = pltpu.make_async_remote_copy(
                    src_ref=comm.at[h],
                    dst_ref=comm.at[h + 1],
                    send_sem=send_sems.at[h],
                    recv_sem=recv_sems.at[h],
                    device_id=(right,),
                    device_id_type=pl.DeviceIdType.MESH,
                )
                rdma.start()

            kcp.wait()
            vcp.wait()

            wq_c = comm[h, 0]
            q = lax.dot_general(
                xb, wq_c, (((1,), (0,)), ((), ())),
                preferred_element_type=jnp.float32)
            qb = q.astype(jnp.bfloat16)
            kb = k_buf[...].astype(jnp.bfloat16)
            vb = v_buf[...].astype(jnp.bfloat16)

            for hh in range(HQ_PER):
                qh = qb[:, hh * DH:(hh + 1) * DH]
                kh = kb[:, hh, :]
                s = lax.dot_general(
                    qh, kh, (((1,), (1,)), ((), ())),
                    preferred_element_type=jnp.float32)
                s = s * SCALE + mask_buf[...]
                m = jnp.max(s, axis=1, keepdims=True)
                w = jnp.exp(s - m)
                den = jnp.sum(w, axis=1, keepdims=True)
                wb = (w / den).astype(jnp.bfloat16)
                ctx = lax.dot_general(
                    wb, vb[:, hh, :], (((1,), (0,)), ((), ())),
                    preferred_element_type=jnp.float32)
                ctx_buf[:, hh * DH:(hh + 1) * DH] = ctx.astype(jnp.bfloat16)

            contrib = lax.dot_general(
                ctx_buf[...], comm[h, 1], (((1,), (0,)), ((), ())),
                preferred_element_type=jnp.float32)
            if h == 0:
                out_ref[0] = contrib
            else:
                out_ref[0] = out_ref[0] + contrib

            if h < N_DEV - 1:
                rdma.wait()

    return pl.pallas_call(
        body,
        out_shape=jax.ShapeDtypeStruct((1, SQ, D_MODEL), jnp.float32),
        in_specs=[
            pl.BlockSpec(memory_space=pltpu.VMEM),
            pl.BlockSpec(memory_space=pltpu.VMEM),
            pl.BlockSpec(memory_space=pl.ANY),
            pl.BlockSpec(memory_space=pl.ANY),
            pl.BlockSpec(memory_space=pltpu.VMEM),
        ],
        out_specs=pl.BlockSpec(memory_space=pltpu.VMEM),
        scratch_shapes=[
            pltpu.VMEM((N_DEV, 2, CHUNK, D_MODEL), jnp.bfloat16),
            pltpu.VMEM((SKV, HQ_PER, DH), jnp.float32),
            pltpu.VMEM((SKV, HQ_PER, DH), jnp.float32),
            pltpu.VMEM((SQ, CHUNK), jnp.bfloat16),
            pltpu.VMEM((SQ, SKV), jnp.float32),
            pltpu.SemaphoreType.DMA((N_DEV - 1,)),
            pltpu.SemaphoreType.DMA((N_DEV - 1,)),
            pltpu.SemaphoreType.DMA((2,)),
        ],
        compiler_params=pltpu.CompilerParams(
            collective_id=0,
            vmem_limit_bytes=58 * 1024 * 1024,
        ),
    )(x16, wq16, K_ext, V_ext, wo16)


# device time: 124566 ns/iter; 1.5049x vs baseline; 1.5049x over previous
import jax

jax.config.update("jax_compilation_cache_dir", "/tmp/scband_jax_cache")
jax.config.update("jax_persistent_cache_min_compile_time_secs", 0.0)
jax.config.update("jax_persistent_cache_min_entry_size_bytes", 0)

import jax.numpy as jnp
from jax import lax
from jax.experimental import pallas as pl
from jax.experimental.pallas import tpu as pltpu

N_DEV = 4
SQ = 1024
SKV = 1024
D_MODEL = 1024
HQ_PER = 8
HH = HQ_PER // 2
DH = 128
HALF = HH * DH
SCALE = 0.08838834764831843
BLK = 64


def kernel(x, Wq, K_ext, V_ext, Wo):
    x16 = x.astype(jnp.bfloat16)
    wq16 = Wq.astype(jnp.bfloat16)
    wo16 = Wo.astype(jnp.bfloat16)

    def body(x_ref, wq_ref, k_hbm, v_hbm, wo_ref, out_ref,
             cwq, cwo, wwq, wwo,
             kcw, vcw, kww, vww, ctx_cw, ctx_ww, mask_buf,
             ssem, rsem, kv_sems):
        my = lax.axis_index("i")
        right = lax.rem(my + 1, N_DEV)
        left = lax.rem(my + 3, N_DEV)

        barrier = pltpu.get_barrier_semaphore()
        for nbr in (left, right):
            pl.semaphore_signal(
                barrier, inc=1,
                device_id=(nbr,), device_id_type=pl.DeviceIdType.MESH,
            )
        pl.semaphore_wait(barrier, 2)

        cwq[0] = wq_ref[:, :HALF]
        cwo[0] = wo_ref[:HALF, :]
        wwq[0] = wq_ref[:, HALF:]
        wwo[0] = wo_ref[HALF:, :]

        rb = lax.broadcasted_iota(jnp.int32, (SQ, SKV), 0) // BLK
        cb = lax.broadcasted_iota(jnp.int32, (SQ, SKV), 1) // BLK
        mask_buf[...] = jnp.where(cb <= rb, 0.0, -1e9).astype(jnp.bfloat16)

        out_ref[0] = jnp.zeros((SQ, D_MODEL), jnp.float32)
        xb = x_ref[0]

        def step(h, carry):
            c_cw = lax.rem(my - h + N_DEV, N_DEV)
            c_ww = lax.rem(my + h, N_DEV)

            copies = [
                pltpu.make_async_copy(
                    k_hbm.at[my, :, pl.ds(c_cw * HQ_PER, HH), :],
                    kcw, kv_sems.at[0]),
                pltpu.make_async_copy(
                    v_hbm.at[my, :, pl.ds(c_cw * HQ_PER, HH), :],
                    vcw, kv_sems.at[1]),
                pltpu.make_async_copy(
                    k_hbm.at[my, :, pl.ds(c_ww * HQ_PER + HH, HH), :],
                    kww, kv_sems.at[2]),
                pltpu.make_async_copy(
                    v_hbm.at[my, :, pl.ds(c_ww * HQ_PER + HH, HH), :],
                    vww, kv_sems.at[3]),
            ]
            for cp in copies:
                cp.start()

            def rdmas(hh):
                return [
                    pltpu.make_async_remote_copy(
                        src_ref=cwq.at[hh], dst_ref=cwq.at[hh + 1],
                        send_sem=ssem.at[0, hh], recv_sem=rsem.at[0, hh],
                        device_id=(right,),
                        device_id_type=pl.DeviceIdType.MESH),
                    pltpu.make_async_remote_copy(
                        src_ref=cwo.at[hh], dst_ref=cwo.at[hh + 1],
                        send_sem=ssem.at[1, hh], recv_sem=rsem.at[1, hh],
                        device_id=(right,),
                        device_id_type=pl.DeviceIdType.MESH),
                    pltpu.make_async_remote_copy(
                        src_ref=wwq.at[hh], dst_ref=wwq.at[hh + 1],
                        send_sem=ssem.at[2, hh], recv_sem=rsem.at[2, hh],
                        device_id=(left,),
                        device_id_type=pl.DeviceIdType.MESH),
                    pltpu.make_async_remote_copy(
                        src_ref=wwo.at[hh], dst_ref=wwo.at[hh + 1],
                        send_sem=ssem.at[3, hh], recv_sem=rsem.at[3, hh],
                        device_id=(left,),
                        device_id_type=pl.DeviceIdType.MESH),
                ]

            @pl.when(h < N_DEV - 1)
            def _():
                hc = lax.min(h, N_DEV - 2)
                for r in rdmas(hc):
                    r.start()

            for cp in copies:
                cp.wait()

            for qbuf, obuf, kbuf, vbuf, cbuf in (
                (cwq, cwo, kcw, vcw, ctx_cw),
                (wwq, wwo, kww, vww, ctx_ww),
            ):
                wq_h = qbuf[h]
                qb = lax.dot_general(
                    xb, wq_h, (((1,), (0,)), ((), ())),
                    preferred_element_type=jnp.float32,
                ).astype(jnp.bfloat16)
                for i in range(HH):
                    qh = qb[:, i * DH:(i + 1) * DH]
                    kb_i = kbuf[:, i, :].astype(jnp.bfloat16)
                    s = lax.dot_general(
                        qh, kb_i, (((1,), (1,)), ((), ())),
                        preferred_element_type=jnp.float32)
                    w = jnp.exp(
                        (s * SCALE).astype(jnp.bfloat16) + mask_buf[...])
                    den = jnp.sum(w, axis=1, keepdims=True)
                    vb_i = vbuf[:, i, :].astype(jnp.bfloat16)
                    ce = lax.dot_general(
                        w, vb_i, (((1,), (0,)), ((), ())),
                        preferred_element_type=jnp.float32)
                    ctx = ce * (1.0 / den.astype(jnp.float32))
                    cbuf[:, i * DH:(i + 1) * DH] = ctx.astype(jnp.bfloat16)
                contrib = lax.dot_general(
                    cbuf[...], obuf[h], (((1,), (0,)), ((), ())),
                    preferred_element_type=jnp.float32)
                out_ref[0] = out_ref[0] + contrib

            @pl.when(h < N_DEV - 1)
            def _():
                hc = lax.min(h, N_DEV - 2)
                for r in rdmas(hc):
                    r.wait()

            return carry

        lax.fori_loop(0, N_DEV, step, 0)

    return pl.pallas_call(
        body,
        out_shape=jax.ShapeDtypeStruct((1, SQ, D_MODEL), jnp.float32),
        in_specs=[
            pl.BlockSpec(memory_space=pltpu.VMEM),
            pl.BlockSpec(memory_space=pltpu.VMEM),
            pl.BlockSpec(memory_space=pl.ANY),
            pl.BlockSpec(memory_space=pl.ANY),
            pl.BlockSpec(memory_space=pltpu.VMEM),
        ],
        out_specs=pl.BlockSpec(memory_space=pltpu.VMEM),
        scratch_shapes=[
            pltpu.VMEM((N_DEV, D_MODEL, HALF), jnp.bfloat16),
            pltpu.VMEM((N_DEV, HALF, D_MODEL), jnp.bfloat16),
            pltpu.VMEM((N_DEV, D_MODEL, HALF), jnp.bfloat16),
            pltpu.VMEM((N_DEV, HALF, D_MODEL), jnp.bfloat16),
            pltpu.VMEM((SKV, HH, DH), jnp.float32),
            pltpu.VMEM((SKV, HH, DH), jnp.float32),
            pltpu.VMEM((SKV, HH, DH), jnp.float32),
            pltpu.VMEM((SKV, HH, DH), jnp.float32),
            pltpu.VMEM((SQ, HALF), jnp.bfloat16),
            pltpu.VMEM((SQ, HALF), jnp.bfloat16),
            pltpu.VMEM((SQ, SKV), jnp.bfloat16),
            pltpu.SemaphoreType.DMA((4, N_DEV - 1)),
            pltpu.SemaphoreType.DMA((4, N_DEV - 1)),
            pltpu.SemaphoreType.DMA((4,)),
        ],
        compiler_params=pltpu.CompilerParams(
            collective_id=0,
            vmem_limit_bytes=58 * 1024 * 1024,
        ),
    )(x16, wq16, K_ext, V_ext, wo16)


# device time: 111542 ns/iter; 1.6807x vs baseline; 1.1168x over previous
import jax

jax.config.update("jax_compilation_cache_dir", "/tmp/scband_jax_cache")
jax.config.update("jax_persistent_cache_min_compile_time_secs", 0.0)
jax.config.update("jax_persistent_cache_min_entry_size_bytes", 0)

import jax.numpy as jnp
from jax import lax
from jax.experimental import pallas as pl
from jax.experimental.pallas import tpu as pltpu

N_DEV = 4
SQ = 1024
SKV = 1024
D_MODEL = 1024
HQ_PER = 8
HH = HQ_PER // 2
DH = 128
HALF = HH * DH
SCALE = 0.08838834764831843
BLK = 64
TS = 256
T = SQ // TS


def kernel(x, Wq, K_ext, V_ext, Wo):
    x16 = x.astype(jnp.bfloat16)
    wq16 = Wq.astype(jnp.bfloat16)
    wo16 = Wo.astype(jnp.bfloat16)

    def body(x_ref, wq_ref, k_hbm, v_hbm, wo_ref, out_ref,
             cwq, cwo, wwq, wwo,
             kcw, vcw, kww, vww, ctx3, mask_buf,
             ssem, rsem, kv_sems):
        my = lax.axis_index("i")
        right = lax.rem(my + 1, N_DEV)
        left = lax.rem(my + 3, N_DEV)

        barrier = pltpu.get_barrier_semaphore()
        for nbr in (left, right):
            pl.semaphore_signal(
                barrier, inc=1,
                device_id=(nbr,), device_id_type=pl.DeviceIdType.MESH,
            )
        pl.semaphore_wait(barrier, 2)

        cwq[0] = wq_ref[:, :HALF]
        cwo[0] = wo_ref[:HALF, :]
        wwq[0] = wq_ref[:, HALF:]
        wwo[0] = wo_ref[HALF:, :]

        rb = lax.broadcasted_iota(jnp.int32, (TS, TS), 0) // BLK
        cb = lax.broadcasted_iota(jnp.int32, (TS, TS), 1) // BLK
        mask_buf[...] = jnp.where(cb <= rb, 0.0, -1e9)

        out_ref[0] = jnp.zeros((SQ, D_MODEL), jnp.float32)
        xb = x_ref[0]

        def step(h, carry):
            c_cw = lax.rem(my - h + N_DEV, N_DEV)
            c_ww = lax.rem(my + h, N_DEV)

            copies = []
            for i in range(HH):
                copies += [
                    pltpu.make_async_copy(
                        k_hbm.at[my, :, c_cw * HQ_PER + i, :],
                        kcw.at[i], kv_sems.at[i]),
                    pltpu.make_async_copy(
                        v_hbm.at[my, :, c_cw * HQ_PER + i, :],
                        vcw.at[i], kv_sems.at[HH + i]),
                    pltpu.make_async_copy(
                        k_hbm.at[my, :, c_ww * HQ_PER + HH + i, :],
                        kww.at[i], kv_sems.at[2 * HH + i]),
                    pltpu.make_async_copy(
                        v_hbm.at[my, :, c_ww * HQ_PER + HH + i, :],
                        vww.at[i], kv_sems.at[3 * HH + i]),
                ]
            for cp in copies:
                cp.start()

            def rdmas(hh):
                return [
                    pltpu.make_async_remote_copy(
                        src_ref=cwq.at[hh], dst_ref=cwq.at[hh + 1],
                        send_sem=ssem.at[0, hh], recv_sem=rsem.at[0, hh],
                        device_id=(right,),
                        device_id_type=pl.DeviceIdType.MESH),
                    pltpu.make_async_remote_copy(
                        src_ref=cwo.at[hh], dst_ref=cwo.at[hh + 1],
                        send_sem=ssem.at[1, hh], recv_sem=rsem.at[1, hh],
                        device_id=(right,),
                        device_id_type=pl.DeviceIdType.MESH),
                    pltpu.make_async_remote_copy(
                        src_ref=wwq.at[hh], dst_ref=wwq.at[hh + 1],
                        send_sem=ssem.at[2, hh], recv_sem=rsem.at[2, hh],
                        device_id=(left,),
                        device_id_type=pl.DeviceIdType.MESH),
                    pltpu.make_async_remote_copy(
                        src_ref=wwo.at[hh], dst_ref=wwo.at[hh + 1],
                        send_sem=ssem.at[3, hh], recv_sem=rsem.at[3, hh],
                        device_id=(left,),
                        device_id_type=pl.DeviceIdType.MESH),
                ]

            @pl.when(h < N_DEV - 1)
            def _():
                hc = lax.min(h, N_DEV - 2)
                for r in rdmas(hc):
                    r.start()

            for cp in copies:
                cp.wait()

            bn = (((2,), (2,)), ((0,), (0,)))
            bv = (((2,), (1,)), ((0,), (0,)))
            for qbuf, obuf, kbuf, vbuf in (
                (cwq, cwo, kcw, vcw),
                (wwq, wwo, kww, vww),
            ):
                wq_h = qbuf[h]
                q = lax.dot_general(
                    xb, wq_h, (((1,), (0,)), ((), ())),
                    preferred_element_type=jnp.float32)
                q3 = q.reshape(SQ, HH, DH).transpose(1, 0, 2)
                for t in range(T):
                    r0 = t * TS
                    qt = q3[:, r0:r0 + TS, :]
                    sd = lax.dot_general(
                        qt, kbuf[:, r0:r0 + TS, :], bn,
                        preferred_element_type=jnp.float32)
                    wd = jnp.exp(sd * SCALE + mask_buf[...][None, :, :])
                    den = jnp.sum(wd, axis=2, keepdims=True)
                    ce = lax.dot_general(
                        wd, vbuf[:, r0:r0 + TS, :], bv,
                        preferred_element_type=jnp.float32)
                    if t > 0:
                        sf = lax.dot_general(
                            qt, kbuf[:, :r0, :], bn,
                            preferred_element_type=jnp.float32)
                        wf = jnp.exp(sf * SCALE)
                        den = den + jnp.sum(wf, axis=2, keepdims=True)
                        ce = ce + lax.dot_general(
                            wf, vbuf[:, :r0, :], bv,
                            preferred_element_type=jnp.float32)
                    ctx3[:, r0:r0 + TS, :] = ce / den
                ctxt = (
                    ctx3[...].transpose(1, 0, 2).reshape(SQ, HALF)
                ).astype(jnp.bfloat16)
                contrib = lax.dot_general(
                    ctxt, obuf[h], (((1,), (0,)), ((), ())),
                    preferred_element_type=jnp.float32)
                out_ref[0] = out_ref[0] + contrib

            @pl.when(h < N_DEV - 1)
            def _():
                hc = lax.min(h, N_DEV - 2)
                for r in rdmas(hc):
                    r.wait()

            return carry

        lax.fori_loop(0, N_DEV, step, 0)

    return pl.pallas_call(
        body,
        out_shape=jax.ShapeDtypeStruct((1, SQ, D_MODEL), jnp.float32),
        in_specs=[
            pl.BlockSpec(memory_space=pltpu.VMEM),
            pl.BlockSpec(memory_space=pltpu.VMEM),
            pl.BlockSpec(memory_space=pl.ANY),
            pl.BlockSpec(memory_space=pl.ANY),
            pl.BlockSpec(memory_space=pltpu.VMEM),
        ],
        out_specs=pl.BlockSpec(memory_space=pltpu.VMEM),
        scratch_shapes=[
            pltpu.VMEM((N_DEV, D_MODEL, HALF), jnp.bfloat16),
            pltpu.VMEM((N_DEV, HALF, D_MODEL), jnp.bfloat16),
            pltpu.VMEM((N_DEV, D_MODEL, HALF), jnp.bfloat16),
            pltpu.VMEM((N_DEV, HALF, D_MODEL), jnp.bfloat16),
            pltpu.VMEM((HH, SKV, DH), jnp.float32),
            pltpu.VMEM((HH, SKV, DH), jnp.float32),
            pltpu.VMEM((HH, SKV, DH), jnp.float32),
            pltpu.VMEM((HH, SKV, DH), jnp.float32),
            pltpu.VMEM((HH, SQ, DH), jnp.float32),
            pltpu.VMEM((TS, TS), jnp.float32),
            pltpu.SemaphoreType.DMA((4, N_DEV - 1)),
            pltpu.SemaphoreType.DMA((4, N_DEV - 1)),
            pltpu.SemaphoreType.DMA((4 * HH,)),
        ],
        compiler_params=pltpu.CompilerParams(
            collective_id=0,
            vmem_limit_bytes=58 * 1024 * 1024,
        ),
    )(x16, wq16, K_ext, V_ext, wo16)


# device time: 107808 ns/iter; 1.7389x vs baseline; 1.0346x over previous
import jax

jax.config.update("jax_compilation_cache_dir", "/tmp/scband_jax_cache")
jax.config.update("jax_persistent_cache_min_compile_time_secs", 0.0)
jax.config.update("jax_persistent_cache_min_entry_size_bytes", 0)

import jax.numpy as jnp
from jax import lax
from jax.experimental import pallas as pl
from jax.experimental.pallas import tpu as pltpu

N_DEV = 4
SQ = 1024
SKV = 1024
D_MODEL = 1024
HQ_PER = 8
HH = HQ_PER // 2
DH = 128
HALF = HH * DH
SCALE = 0.08838834764831843
BLK = 64
TS = 256
T = SQ // TS


def kernel(x, Wq, K_ext, V_ext, Wo):
    x16 = x.astype(jnp.bfloat16)
    wq16 = Wq.astype(jnp.bfloat16)
    wo16 = Wo.astype(jnp.bfloat16)

    def body(x_ref, wq_ref, k_hbm, v_hbm, wo_ref, out_ref,
             cwq, cwo, wwq, wwo,
             kcw, vcw, kww, vww, ctx3, mask_buf,
             ssem, rsem, kv_sems):
        my = lax.axis_index("i")
        right = lax.rem(my + 1, N_DEV)
        left = lax.rem(my + 3, N_DEV)

        barrier = pltpu.get_barrier_semaphore()
        for nbr in (left, right):
            pl.semaphore_signal(
                barrier, inc=1,
                device_id=(nbr,), device_id_type=pl.DeviceIdType.MESH,
            )
        pl.semaphore_wait(barrier, 2)

        cwq[0] = wq_ref[:, :HALF]
        cwo[0] = wo_ref[:HALF, :]
        wwq[0] = wq_ref[:, HALF:]
        wwo[0] = wo_ref[HALF:, :]

        rb = lax.broadcasted_iota(jnp.int32, (TS, TS), 0) // BLK
        cb = lax.broadcasted_iota(jnp.int32, (TS, TS), 1) // BLK
        mask_buf[...] = jnp.where(cb <= rb, 0.0, -1e9)

        out_ref[0] = jnp.zeros((SQ, D_MODEL), jnp.float32)
        xb = x_ref[0]

        def kv_copies(s, p):
            c_cw = lax.rem(my - s + N_DEV, N_DEV)
            c_ww = lax.rem(my + s, N_DEV)
            copies = []
            for i in range(HH):
                copies += [
                    pltpu.make_async_copy(
                        k_hbm.at[my, :, c_cw * HQ_PER + i, :],
                        kcw.at[p, i], kv_sems.at[p, i]),
                    pltpu.make_async_copy(
                        v_hbm.at[my, :, c_cw * HQ_PER + i, :],
                        vcw.at[p, i], kv_sems.at[p, HH + i]),
                    pltpu.make_async_copy(
                        k_hbm.at[my, :, c_ww * HQ_PER + HH + i, :],
                        kww.at[p, i], kv_sems.at[p, 2 * HH + i]),
                    pltpu.make_async_copy(
                        v_hbm.at[my, :, c_ww * HQ_PER + HH + i, :],
                        vww.at[p, i], kv_sems.at[p, 3 * HH + i]),
                ]
            return copies

        for cp in kv_copies(0, 0):
            cp.start()

        def step(h, carry):
            par = lax.rem(h, 2)

            def rdmas(hh):
                return [
                    pltpu.make_async_remote_copy(
                        src_ref=cwq.at[hh], dst_ref=cwq.at[hh + 1],
                        send_sem=ssem.at[0, hh], recv_sem=rsem.at[0, hh],
                        device_id=(right,),
                        device_id_type=pl.DeviceIdType.MESH),
                    pltpu.make_async_remote_copy(
                        src_ref=cwo.at[hh], dst_ref=cwo.at[hh + 1],
                        send_sem=ssem.at[1, hh], recv_sem=rsem.at[1, hh],
                        device_id=(right,),
                        device_id_type=pl.DeviceIdType.MESH),
                    pltpu.make_async_remote_copy(
                        src_ref=wwq.at[hh], dst_ref=wwq.at[hh + 1],
                        send_sem=ssem.at[2, hh], recv_sem=rsem.at[2, hh],
                        device_id=(left,),
                        device_id_type=pl.DeviceIdType.MESH),
                    pltpu.make_async_remote_copy(
                        src_ref=wwo.at[hh], dst_ref=wwo.at[hh + 1],
                        send_sem=ssem.at[3, hh], recv_sem=rsem.at[3, hh],
                        device_id=(left,),
                        device_id_type=pl.DeviceIdType.MESH),
                ]

            @pl.when(h < N_DEV - 1)
            def _():
                hc = lax.min(h, N_DEV - 2)
                for r in rdmas(hc):
                    r.start()

            @pl.when(h < N_DEV - 1)
            def _():
                for cp in kv_copies(h + 1, 1 - par):
                    cp.start()

            for cp in kv_copies(h, par):
                cp.wait()

            bn = (((2,), (2,)), ((0,), (0,)))
            bv = (((2,), (1,)), ((0,), (0,)))
            for qbuf, obuf, kbuf, vbuf in (
                (cwq, cwo, kcw, vcw),
                (wwq, wwo, kww, vww),
            ):
                wq_h = qbuf[h]
                q = lax.dot_general(
                    xb, wq_h, (((1,), (0,)), ((), ())),
                    preferred_element_type=jnp.float32)
                q3 = q.reshape(SQ, HH, DH).transpose(1, 0, 2)
                for t in range(T):
                    r0 = t * TS
                    qt = q3[:, r0:r0 + TS, :]
                    sd = lax.dot_general(
                        qt, kbuf[par, :, r0:r0 + TS, :], bn,
                        preferred_element_type=jnp.float32)
                    wd = jnp.exp(sd * SCALE + mask_buf[...][None, :, :])
                    den = jnp.sum(wd, axis=2, keepdims=True)
                    ce = lax.dot_general(
                        wd, vbuf[par, :, r0:r0 + TS, :], bv,
                        preferred_element_type=jnp.float32)
                    if t > 0:
                        sf = lax.dot_general(
                            qt, kbuf[par, :, :r0, :], bn,
                            preferred_element_type=jnp.float32)
                        wf = jnp.exp(sf * SCALE)
                        den = den + jnp.sum(wf, axis=2, keepdims=True)
                        ce = ce + lax.dot_general(
                            wf, vbuf[par, :, :r0, :], bv,
                            preferred_element_type=jnp.float32)
                    ctx3[:, r0:r0 + TS, :] = ce / den
                ctxt = (
                    ctx3[...].transpose(1, 0, 2).reshape(SQ, HALF)
                ).astype(jnp.bfloat16)
                contrib = lax.dot_general(
                    ctxt, obuf[h], (((1,), (0,)), ((), ())),
                    preferred_element_type=jnp.float32)
                out_ref[0] = out_ref[0] + contrib

            @pl.when(h < N_DEV - 1)
            def _():
                hc = lax.min(h, N_DEV - 2)
                for r in rdmas(hc):
                    r.wait()

            return carry

        lax.fori_loop(0, N_DEV, step, 0)

    return pl.pallas_call(
        body,
        out_shape=jax.ShapeDtypeStruct((1, SQ, D_MODEL), jnp.float32),
        in_specs=[
            pl.BlockSpec(memory_space=pltpu.VMEM),
            pl.BlockSpec(memory_space=pltpu.VMEM),
            pl.BlockSpec(memory_space=pl.ANY),
            pl.BlockSpec(memory_space=pl.ANY),
            pl.BlockSpec(memory_space=pltpu.VMEM),
        ],
        out_specs=pl.BlockSpec(memory_space=pltpu.VMEM),
        scratch_shapes=[
            pltpu.VMEM((N_DEV, D_MODEL, HALF), jnp.bfloat16),
            pltpu.VMEM((N_DEV, HALF, D_MODEL), jnp.bfloat16),
            pltpu.VMEM((N_DEV, D_MODEL, HALF), jnp.bfloat16),
            pltpu.VMEM((N_DEV, HALF, D_MODEL), jnp.bfloat16),
            pltpu.VMEM((2, HH, SKV, DH), jnp.float32),
            pltpu.VMEM((2, HH, SKV, DH), jnp.float32),
            pltpu.VMEM((2, HH, SKV, DH), jnp.float32),
            pltpu.VMEM((2, HH, SKV, DH), jnp.float32),
            pltpu.VMEM((HH, SQ, DH), jnp.float32),
            pltpu.VMEM((TS, TS), jnp.float32),
            pltpu.SemaphoreType.DMA((4, N_DEV - 1)),
            pltpu.SemaphoreType.DMA((4, N_DEV - 1)),
            pltpu.SemaphoreType.DMA((2, 4 * HH)),
        ],
        compiler_params=pltpu.CompilerParams(
            collective_id=0,
            vmem_limit_bytes=58 * 1024 * 1024,
        ),
    )(x16, wq16, K_ext, V_ext, wo16)


# device time: 107375 ns/iter; 1.7459x vs baseline; 1.0040x over previous
import jax

jax.config.update("jax_compilation_cache_dir", "/tmp/scband_jax_cache")
jax.config.update("jax_persistent_cache_min_compile_time_secs", 0.0)
jax.config.update("jax_persistent_cache_min_entry_size_bytes", 0)

import jax.numpy as jnp
from jax import lax
from jax.experimental import pallas as pl
from jax.experimental.pallas import tpu as pltpu

N_DEV = 4
SQ = 1024
SKV = 1024
D_MODEL = 1024
HQ_PER = 8
HH = HQ_PER // 2
DH = 128
HALF = HH * DH
SCALE = 0.08838834764831843
BLK = 64
TS = 256
T = SQ // TS


def kernel(x, Wq, K_ext, V_ext, Wo):
    x16 = x.astype(jnp.bfloat16)
    wq16 = Wq.astype(jnp.bfloat16)
    wo16 = Wo.astype(jnp.bfloat16)

    def body(x_ref, wq_ref, k_hbm, v_hbm, wo_ref, out_ref,
             cwq, cwo, wwq, wwo,
             kcw, vcw, kww, vww, ctx3, mask_buf,
             ssem, rsem, kv_sems):
        my = lax.axis_index("i")
        right = lax.rem(my + 1, N_DEV)
        left = lax.rem(my + 3, N_DEV)

        barrier = pltpu.get_barrier_semaphore()
        for nbr in (left, right):
            pl.semaphore_signal(
                barrier, inc=1,
                device_id=(nbr,), device_id_type=pl.DeviceIdType.MESH,
            )
        pl.semaphore_wait(barrier, 2)

        cwq[0] = wq_ref[:, :HALF]
        cwo[0] = wo_ref[:HALF, :]
        wwq[0] = wq_ref[:, HALF:]
        wwo[0] = wo_ref[HALF:, :]

        rb = lax.broadcasted_iota(jnp.int32, (TS, TS), 0) // BLK
        cb = lax.broadcasted_iota(jnp.int32, (TS, TS), 1) // BLK
        mask_buf[...] = jnp.where(cb <= rb, 0.0, -1e9)

        out_ref[0] = jnp.zeros((SQ, D_MODEL), jnp.float32)
        xb = x_ref[0]

        def kv_copies(s, p):
            c_cw = lax.rem(my - s + N_DEV, N_DEV)
            c_ww = lax.rem(my + s, N_DEV)
            copies = []
            for i in range(HH):
                copies += [
                    pltpu.make_async_copy(
                        k_hbm.at[my, :, c_cw * HQ_PER + i, :],
                        kcw.at[p, i], kv_sems.at[p, i]),
                    pltpu.make_async_copy(
                        v_hbm.at[my, :, c_cw * HQ_PER + i, :],
                        vcw.at[p, i], kv_sems.at[p, HH + i]),
                    pltpu.make_async_copy(
                        k_hbm.at[my, :, c_ww * HQ_PER + HH + i, :],
                        kww.at[p, i], kv_sems.at[p, 2 * HH + i]),
                    pltpu.make_async_copy(
                        v_hbm.at[my, :, c_ww * HQ_PER + HH + i, :],
                        vww.at[p, i], kv_sems.at[p, 3 * HH + i]),
                ]
            return copies

        for cp in kv_copies(0, 0):
            cp.start()

        def step(h, carry):
            par = lax.rem(h, 2)

            def rdmas(hh):
                return [
                    pltpu.make_async_remote_copy(
                        src_ref=cwq.at[hh], dst_ref=cwq.at[hh + 1],
                        send_sem=ssem.at[0, hh], recv_sem=rsem.at[0, hh],
                        device_id=(right,),
                        device_id_type=pl.DeviceIdType.MESH),
                    pltpu.make_async_remote_copy(
                        src_ref=cwo.at[hh], dst_ref=cwo.at[hh + 1],
                        send_sem=ssem.at[1, hh], recv_sem=rsem.at[1, hh],
                        device_id=(right,),
                        device_id_type=pl.DeviceIdType.MESH),
                    pltpu.make_async_remote_copy(
                        src_ref=wwq.at[hh], dst_ref=wwq.at[hh + 1],
                        send_sem=ssem.at[2, hh], recv_sem=rsem.at[2, hh],
                        device_id=(left,),
                        device_id_type=pl.DeviceIdType.MESH),
                    pltpu.make_async_remote_copy(
                        src_ref=wwo.at[hh], dst_ref=wwo.at[hh + 1],
                        send_sem=ssem.at[3, hh], recv_sem=rsem.at[3, hh],
                        device_id=(left,),
                        device_id_type=pl.DeviceIdType.MESH),
                ]

            @pl.when(h < N_DEV - 1)
            def _():
                hc = lax.min(h, N_DEV - 2)
                for r in rdmas(hc):
                    r.start()

            @pl.when(h < N_DEV - 1)
            def _():
                for cp in kv_copies(h + 1, 1 - par):
                    cp.start()

            for cp in kv_copies(h, par):
                cp.wait()

            ck = (((1,), (1,)), ((), ()))
            cv = (((1,), (0,)), ((), ()))
            for qbuf, obuf, kbuf, vbuf in (
                (cwq, cwo, kcw, vcw),
                (wwq, wwo, kww, vww),
            ):
                wq_h = qbuf[h]
                q = lax.dot_general(
                    xb, wq_h, (((1,), (0,)), ((), ())),
                    preferred_element_type=jnp.float32)
                for i in range(HH):
                    for t in range(T):
                        r0 = t * TS
                        qt = q[r0:r0 + TS, i * DH:(i + 1) * DH]
                        sd = lax.dot_general(
                            qt, kbuf[par, i, r0:r0 + TS, :], ck,
                            preferred_element_type=jnp.float32)
                        wd = jnp.exp(sd * SCALE + mask_buf[...])
                        den = jnp.sum(wd, axis=1, keepdims=True)
                        ce = lax.dot_general(
                            wd, vbuf[par, i, r0:r0 + TS, :], cv,
                            preferred_element_type=jnp.float32)
                        if t > 0:
                            sf = lax.dot_general(
                                qt, kbuf[par, i, :r0, :], ck,
                                preferred_element_type=jnp.float32)
                            wf = jnp.exp(sf * SCALE)
                            den = den + jnp.sum(wf, axis=1, keepdims=True)
                            ce = ce + lax.dot_general(
                                wf, vbuf[par, i, :r0, :], cv,
                                preferred_element_type=jnp.float32)
                        ctx3[r0:r0 + TS, i * DH:(i + 1) * DH] = ce / den
                ctxt = ctx3[...].astype(jnp.bfloat16)
                contrib = lax.dot_general(
                    ctxt, obuf[h], (((1,), (0,)), ((), ())),
                    preferred_element_type=jnp.float32)
                out_ref[0] = out_ref[0] + contrib

            @pl.when(h < N_DEV - 1)
            def _():
                hc = lax.min(h, N_DEV - 2)
                for r in rdmas(hc):
                    r.wait()

            return carry

        lax.fori_loop(0, N_DEV, step, 0)

    return pl.pallas_call(
        body,
        out_shape=jax.ShapeDtypeStruct((1, SQ, D_MODEL), jnp.float32),
        in_specs=[
            pl.BlockSpec(memory_space=pltpu.VMEM),
            pl.BlockSpec(memory_space=pltpu.VMEM),
            pl.BlockSpec(memory_space=pl.ANY),
            pl.BlockSpec(memory_space=pl.ANY),
            pl.BlockSpec(memory_space=pltpu.VMEM),
        ],
        out_specs=pl.BlockSpec(memory_space=pltpu.VMEM),
        scratch_shapes=[
            pltpu.VMEM((N_DEV, D_MODEL, HALF), jnp.bfloat16),
            pltpu.VMEM((N_DEV, HALF, D_MODEL), jnp.bfloat16),
            pltpu.VMEM((N_DEV, D_MODEL, HALF), jnp.bfloat16),
            pltpu.VMEM((N_DEV, HALF, D_MODEL), jnp.bfloat16),
            pltpu.VMEM((2, HH, SKV, DH), jnp.float32),
            pltpu.VMEM((2, HH, SKV, DH), jnp.float32),
            pltpu.VMEM((2, HH, SKV, DH), jnp.float32),
            pltpu.VMEM((2, HH, SKV, DH), jnp.float32),
            pltpu.VMEM((SQ, HALF), jnp.float32),
            pltpu.VMEM((TS, TS), jnp.float32),
            pltpu.SemaphoreType.DMA((4, N_DEV - 1)),
            pltpu.SemaphoreType.DMA((4, N_DEV - 1)),
            pltpu.SemaphoreType.DMA((2, 4 * HH)),
        ],
        compiler_params=pltpu.CompilerParams(
            collective_id=0,
            vmem_limit_bytes=58 * 1024 * 1024,
        ),
    )(x16, wq16, K_ext, V_ext, wo16)
